# baseline (speedup 1.0000x reference)
import functools
import jax
import jax.numpy as jnp
from jax import lax
from jax.experimental import pallas as pl
from jax.experimental.pallas import tpu as pltpu
from jax.experimental.pallas import tpu_sc as plsc

N_ENT = 10000
N_REL = 500
D = 200
BN = 20000
EB = 160000
EG = 160000
B = 4096
NHID = 256
HID = 512

_I32 = jnp.int32
_F32 = jnp.float32


@functools.lru_cache(maxsize=None)
def _make_prop_kernel(nc, C, e_pad, n_dst, two_src):
    EBLK = e_pad // 2048
    n_out = -(-n_dst // 128) * 128
    n_zpad = n_out if n_dst < n_out else n_out + 128
    rz = n_zpad // 16
    ro = n_out // 16
    mesh = plsc.VectorSubcoreMesh(core_axis_name="c", subcore_axis_name="s")
    scratch = [
        pltpu.VMEM((EBLK, 128), _I32),
        pltpu.VMEM((EBLK, 128), _I32),
        pltpu.VMEM((EBLK, 128), _I32),
        pltpu.VMEM((128, C), _F32),
        pltpu.VMEM_SHARED((n_zpad, C), _F32),
        pltpu.SemaphoreType.DMA,
    ]

    @functools.partial(
        pl.kernel, mesh=mesh,
        out_type=jax.ShapeDtypeStruct((nc * n_out, C), _F32),
        compiler_params=pltpu.CompilerParams(use_tc_tiling_on_sc=False),
        scratch_types=scratch)
    def k(*refs):
        tabs = refs[:nc]
        (srcA, srcB, dst2d, zrows, out,
         srcA_v, srcB_v, dst_v, buf, acc, sem) = refs[nc:]
        tid = lax.axis_index("s")
        sc = lax.axis_index("c")
        pltpu.sync_copy(dst2d.at[tid], dst_v)
        pltpu.sync_copy(srcA.at[tid], srcA_v)
        if two_src:
            pltpu.sync_copy(srcB.at[tid], srcB_v)
        for ci in range(nc):
            @pl.when(sc == ci % 2)
            def _(ci=ci):
                pltpu.sync_copy(zrows.at[pl.ds(tid * rz, rz)],
                                acc.at[pl.ds(tid * rz, rz)])
                plsc.subcore_barrier()
                tab = tabs[ci]
                s_v = srcB_v if (two_src and ci >= nc // 2) else srcA_v

                def step(j, carry):
                    pltpu.async_copy(tab.at[s_v.at[j]], buf, sem).wait()
                    pltpu.sync_copy(buf, acc.at[dst_v.at[j]], add=True)
                    return carry

                lax.fori_loop(0, EBLK, step, 0)
                plsc.subcore_barrier()
                pltpu.sync_copy(acc.at[pl.ds(tid * ro, ro)],
                                out.at[pl.ds(ci * n_out + tid * ro, ro)])
                plsc.subcore_barrier()

    return k


def _prop(tabs, srcA, srcB, dst, n_dst, two_src=False):
    nc = len(tabs)
    C = tabs[0].shape[1]
    e_pad = dst.shape[0]
    n_out = -(-n_dst // 128) * 128
    k = _make_prop_kernel(nc, C, e_pad, n_dst, two_src)
    eblk = e_pad // 2048
    srcA2 = srcA.reshape(16, eblk, 128)
    srcB2 = srcB.reshape(16, eblk, 128)
    dst2 = dst.reshape(16, eblk, 128)
    zrows = jnp.zeros((n_out if n_dst < n_out else n_out + 128, C), _F32)
    raw = k(*tabs, srcA2, srcB2, dst2, zrows)
    return [raw[i * n_out:i * n_out + n_dst] for i in range(nc)]


@functools.lru_cache(maxsize=None)
def _make_count_kernel(e_pad, n_dst):
    EBLK = e_pad // 4096
    n_out = -(-n_dst // 128) * 128
    n_zpad = n_out if n_dst < n_out else n_out + 128
    rz = n_zpad // 16
    ro = n_out // 16
    mesh = plsc.VectorSubcoreMesh(core_axis_name="c", subcore_axis_name="s")
    scratch = [
        pltpu.VMEM((EBLK, 128), _I32),
        pltpu.VMEM((128, 16), _F32),
        pltpu.VMEM_SHARED((n_zpad, 16), _F32),
    ]

    @functools.partial(
        pl.kernel, mesh=mesh,
        out_type=jax.ShapeDtypeStruct((2 * n_out, 16), _F32),
        compiler_params=pltpu.CompilerParams(use_tc_tiling_on_sc=False),
        scratch_types=scratch)
    def k(dst2d, zrows, ones_hbm, out, dst_v, buf, acc):
        tid = lax.axis_index("s")
        sc = lax.axis_index("c")
        g = sc * 16 + tid
        pltpu.sync_copy(dst2d.at[g], dst_v)
        pltpu.sync_copy(ones_hbm, buf)
        pltpu.sync_copy(zrows.at[pl.ds(tid * rz, rz)],
                        acc.at[pl.ds(tid * rz, rz)])
        plsc.subcore_barrier()

        def step(j, carry):
            pltpu.sync_copy(buf, acc.at[dst_v.at[j]], add=True)
            return carry

        lax.fori_loop(0, EBLK, step, 0)
        plsc.subcore_barrier()
        pltpu.sync_copy(acc.at[pl.ds(tid * ro, ro)],
                        out.at[pl.ds(sc * n_out + tid * ro, ro)])

    return k


def _seg_count(dst, n_dst):
    E = dst.shape[0]
    e_pad = -(-E // 4096) * 4096
    dstp = jnp.concatenate([dst, jnp.full((e_pad - E,), n_dst, _I32)])
    n_out = -(-n_dst // 128) * 128
    k = _make_count_kernel(e_pad, n_dst)
    ones = jnp.ones((128, 16), _F32)
    zrows = jnp.zeros((n_out if n_dst < n_out else n_out + 128, 16), _F32)
    po = k(dstp.reshape(32, e_pad // 4096, 128), zrows, ones)
    return po[:n_dst, 0] + po[n_out:n_out + n_dst, 0]


def _pad_e(x, e_pad, fill):
    return jnp.concatenate([x, jnp.full((e_pad - x.shape[0],), fill, _I32)])


def _chunks(x, C):
    n, d = x.shape
    return [x[:, i * C:(i + 1) * C] for i in range(d // C)]


def _row_spec(blk, d):
    return pl.BlockSpec((blk, d), lambda i: (i, 0))


def _full_spec(shape):
    return pl.BlockSpec(shape, lambda i: tuple(0 for _ in shape))


def _norm_body(x_ref, o_ref):
    x = x_ref[...]
    n = jnp.sqrt(jnp.sum(x * x, axis=1, keepdims=True))
    o_ref[...] = x / (n + 1e-12)


def _norm(x):
    n, d = x.shape
    blk = min(n, 1000)
    return pl.pallas_call(
        _norm_body, grid=(n // blk,),
        in_specs=[_row_spec(blk, d)],
        out_specs=_row_spec(blk, d),
        out_shape=jax.ShapeDtypeStruct((n, d), _F32),
    )(x)


def _mm_body(x_ref, w_ref, o_ref):
    o_ref[...] = jax.lax.dot(x_ref[...], w_ref[...],
                             precision=jax.lax.Precision.HIGHEST)


def _mm(x, w):
    n, kdim = x.shape
    m = w.shape[1]
    blk = 1000 if n % 1000 == 0 else n
    return pl.pallas_call(
        _mm_body, grid=(n // blk,),
        in_specs=[_row_spec(blk, kdim), _full_spec((kdim, m))],
        out_specs=_row_spec(blk, m),
        out_shape=jax.ShapeDtypeStruct((n, m), _F32),
    )(x, w)


def _stage_mce1_body(c0, c1, c2, c3, c4, c5, c6, c7, inv, w, *outs):
    h1 = jax.nn.relu(jnp.concatenate(
        [c0[...], c1[...], c2[...], c3[...]], axis=1) * inv[...])
    h2 = jax.nn.relu(jnp.concatenate(
        [c4[...], c5[...], c6[...], c7[...]], axis=1) * inv[...])
    o1 = jax.lax.dot(h1, w[...], precision=jax.lax.Precision.HIGHEST)
    o2 = jax.lax.dot(h2, w[...], precision=jax.lax.Precision.HIGHEST)
    for j in range(3):
        outs[j][...] = o1[:, j * 80:(j + 1) * 80]
        outs[3 + j][...] = o2[:, j * 80:(j + 1) * 80]


def _stage_mce1(p1c, inv_deg, W2p):
    blk = 1000
    return pl.pallas_call(
        _stage_mce1_body, grid=(BN // blk,),
        in_specs=[_row_spec(blk, 64)] * 8 + [_row_spec(blk, 1),
                                             _full_spec((NHID, 240))],
        out_specs=[_row_spec(blk, 80)] * 6,
        out_shape=[jax.ShapeDtypeStruct((BN, 80), _F32)] * 6,
    )(*p1c, inv_deg, W2p)


def _ew_div_body(c0, c1, c2, c3, c4, c5, inv, *outs):
    for j, c in enumerate((c0, c1, c2, c3, c4, c5)):
        outs[j][...] = c[...] * inv[...]


def _ew_div(chunks, inv, n):
    blk = 1000
    return list(pl.pallas_call(
        _ew_div_body, grid=(n // blk,),
        in_specs=[_row_spec(blk, 80)] * 6 + [_row_spec(blk, 1)],
        out_specs=[_row_spec(blk, 80)] * 6,
        out_shape=[jax.ShapeDtypeStruct((n, 80), _F32)] * 6,
    )(*chunks, inv))


def _finish_body(g0, g1, g2, g3, g4, g5, o0, o1, o2, o3, o4, o5,
                 invc, s, wa, wb, bvec, base1, base2,
                 ec, ec_, *e12c):
    gg1 = jnp.concatenate([g0[...], g1[...], g2[...]], axis=1)[:, :D]
    gg2 = jnp.concatenate([g3[...], g4[...], g5[...]], axis=1)[:, :D]
    oo1 = jnp.concatenate([o0[...], o1[...], o2[...]], axis=1)[:, :D] * invc[...]
    oo2 = jnp.concatenate([o3[...], o4[...], o5[...]], axis=1)[:, :D] * invc[...]
    hp = jax.lax.Precision.HIGHEST
    e1 = (s[...] * (jax.lax.dot(gg1, wa[...], precision=hp)
                    + jax.lax.dot(oo1, wb[...], precision=hp))
          + bvec[...] + base1[...])
    e2 = (s[...] * (jax.lax.dot(gg2, wa[...], precision=hp)
                    + jax.lax.dot(oo2, wb[...], precision=hp))
          + bvec[...] + base2[...])
    ec[...] = e1
    ec_[...] = e2
    zpad = jnp.zeros((e1.shape[0], 40), _F32)
    e1p = jnp.concatenate([e1, zpad], axis=1)
    e2p = jnp.concatenate([e2, zpad], axis=1)
    for j in range(3):
        e12c[j][...] = e1p[:, j * 80:(j + 1) * 80]
        e12c[3 + j][...] = e2p[:, j * 80:(j + 1) * 80]


def _finish(g12c, o_raw, inv_cnt, S, leo_W, leo_b, ent, ent_shuf):
    blk = 1000
    outs = pl.pallas_call(
        _finish_body, grid=(N_ENT // blk,),
        in_specs=[_row_spec(blk, 80)] * 12
        + [_row_spec(blk, 1), _row_spec(blk, 1),
           _full_spec((D, D)), _full_spec((D, D)),
           pl.BlockSpec((1, D), lambda i: (0, 0)),
           _row_spec(blk, D), _row_spec(blk, D)],
        out_specs=[_row_spec(blk, D)] * 2 + [_row_spec(blk, 80)] * 6,
        out_shape=[jax.ShapeDtypeStruct((N_ENT, D), _F32)] * 2
        + [jax.ShapeDtypeStruct((N_ENT, 80), _F32)] * 6,
    )(*g12c, *o_raw, inv_cnt, S, leo_W[:D], leo_W[D:], leo_b[None, :],
      ent, ent_shuf)
    return outs[0], outs[1], list(outs[2:])


def _stage_big_body(c0, c1, c2, c3, c4, c5, inv, w1, w2, *outs):
    pb1 = (jnp.concatenate([c0[...], c1[...], c2[...]], axis=1)
           * inv[...])[:, :D]
    pb2 = (jnp.concatenate([c3[...], c4[...], c5[...]], axis=1)
           * inv[...])[:, :D]
    hp = jax.lax.Precision.HIGHEST
    o1 = jax.lax.dot(jax.nn.relu(jax.lax.dot(pb1, w1[...], precision=hp)),
                     w2[...], precision=hp)
    o2 = jax.lax.dot(jax.nn.relu(jax.lax.dot(pb2, w1[...], precision=hp)),
                     w2[...], precision=hp)
    for j in range(3):
        outs[j][...] = o1[:, j * 80:(j + 1) * 80]
        outs[3 + j][...] = o2[:, j * 80:(j + 1) * 80]


def _stage_big(p12_raw, inv_deg_g, W1, W2p):
    blk = 1000
    return list(pl.pallas_call(
        _stage_big_body, grid=(N_ENT // blk,),
        in_specs=[_row_spec(blk, 80)] * 6
        + [_row_spec(blk, 1), _full_spec((D, NHID)), _full_spec((NHID, 240))],
        out_specs=[_row_spec(blk, 80)] * 6,
        out_shape=[jax.ShapeDtypeStruct((N_ENT, 80), _F32)] * 6,
    )(*p12_raw, inv_deg_g, W1, W2p))


def _eg_body(c0, c1, c2, inv, o_ref):
    o_ref[...] = (jnp.concatenate([c0[...], c1[...], c2[...]], axis=1)
                  * inv[...])[:, :D]


def _eg_cat(chunks, inv_deg_g):
    blk = 1000
    return pl.pallas_call(
        _eg_body, grid=(N_ENT // blk,),
        in_specs=[_row_spec(blk, 80)] * 3 + [_row_spec(blk, 1)],
        out_specs=_row_spec(blk, D),
        out_shape=jax.ShapeDtypeStruct((N_ENT, D), _F32),
    )(*chunks, inv_deg_g)


def _dgi_mean_body(h_ref, w_ref, b_ref, o_ref):
    i = pl.program_id(0)
    hp = jax.lax.Precision.HIGHEST
    e = jax.nn.relu(jax.lax.dot(h_ref[...], w_ref[...], precision=hp)
                    + b_ref[...])
    part = jnp.sum(e, axis=0, keepdims=True)

    @pl.when(i == 0)
    def _():
        o_ref[...] = jnp.zeros_like(o_ref)

    o_ref[...] += part


def _dgi_mean(h, W, b):
    blk = 1000
    return pl.pallas_call(
        _dgi_mean_body, grid=(N_ENT // blk,),
        in_specs=[_row_spec(blk, D), _full_spec((D, HID)),
                  pl.BlockSpec((1, HID), lambda i: (0, 0))],
        out_specs=pl.BlockSpec((1, HID), lambda i: (0, 0)),
        out_shape=jax.ShapeDtypeStruct((1, HID), _F32),
    )(h, W, b[None, :])


def _dgi_score_body(h_ref, w_ref, b_ref, v_ref, o_ref):
    hp = jax.lax.Precision.HIGHEST
    e = jax.nn.relu(jax.lax.dot(h_ref[...], w_ref[...], precision=hp)
                    + b_ref[...])
    o_ref[...] = jax.lax.dot(e, v_ref[...], precision=hp)


def _dgi_score(h, W, b, v):
    blk = 1000
    return pl.pallas_call(
        _dgi_score_body, grid=(N_ENT // blk,),
        in_specs=[_row_spec(blk, D), _full_spec((D, HID)),
                  pl.BlockSpec((1, HID), lambda i: (0, 0)),
                  _full_spec((HID, 1))],
        out_specs=_row_spec(blk, 1),
        out_shape=jax.ShapeDtypeStruct((N_ENT, 1), _F32),
    )(h, W, b[None, :], v[:, None])


def _convkb_body(h_ref, r_ref, t_ref, cw_ref, cb_ref, fc2_ref, o_ref):
    h = h_ref[...]
    r = r_ref[...]
    t = t_ref[...]
    acc = jnp.zeros_like(h)
    for o in range(50):
        co = jax.nn.relu(cw_ref[o, 0] * h + cw_ref[o, 1] * r + cw_ref[o, 2] * t
                         + cb_ref[o])
        acc = acc + co * fc2_ref[o, :][None, :]
    o_ref[...] = jnp.sum(acc, axis=1, keepdims=True)


def _convkb(h, r, t, conv_w, conv_b, fc2):
    blk = 1024
    return pl.pallas_call(
        _convkb_body,
        grid=(B // blk,),
        in_specs=[
            pl.BlockSpec((blk, D), lambda i: (i, 0)),
            pl.BlockSpec((blk, D), lambda i: (i, 0)),
            pl.BlockSpec((blk, D), lambda i: (i, 0)),
            pl.BlockSpec((50, 3), lambda i: (0, 0), memory_space=pltpu.SMEM),
            pl.BlockSpec((50,), lambda i: (0,), memory_space=pltpu.SMEM),
            pl.BlockSpec((50, D), lambda i: (0, 0)),
        ],
        out_specs=pl.BlockSpec((blk, 1), lambda i: (i, 0)),
        out_shape=jax.ShapeDtypeStruct((B, 1), jnp.float32),
    )(h, r, t, conv_w, conv_b, fc2)


def kernel(*args):
    with jax.default_matmul_precision("float32"):
        return _kernel_impl(*args)


def _kernel_impl(entity_embeddings, relation_embeddings, sg1_W1, sg1_W2, sg2_W1,
                 sg2_W2, le_W, le_b, leo_W, leo_b, dgi_W, dgi_b, dgi_Wd,
                 conv_w, conv_b, fc_w, fc_b,
                 b_x, b_node_graph_index, b_edge_index, big_edge_index,
                 batch_inputs, shuf_idx):
    ent = _norm(entity_embeddings)
    rel = _norm(relation_embeddings)

    b_x = b_x.astype(_I32)
    brel = b_node_graph_index.astype(_I32)
    src = b_edge_index[0].astype(_I32)
    dst = b_edge_index[1].astype(_I32)
    gsrc = big_edge_index[0].astype(_I32)
    gdst = big_edge_index[1].astype(_I32)
    shuf_idx = shuf_idx.astype(_I32)

    deg_b = jnp.maximum(_seg_count(dst, BN), 1.0)
    deg_g = jnp.maximum(_seg_count(gdst, N_ENT), 1.0)
    cnt = _seg_count(b_x, N_ENT)
    S = cnt / (cnt + 1e-16)
    inv_cnt = 1.0 / jnp.maximum(cnt, 1.0)

    eb_pad = -(-EB // 2048) * 2048
    src_p = _pad_e(src, eb_pad, 0)
    dst_p = _pad_e(dst, eb_pad, BN)
    gsrc_p = _pad_e(gsrc, eb_pad, 0)
    gdst_p = _pad_e(gdst, eb_pad, N_ENT)

    P = _mm(ent, sg1_W1[:D])
    Rr = _mm(rel, sg1_W1[D:])
    sx = jnp.take(shuf_idx, b_x)
    BNP = BN + 480
    ar = jnp.arange(BN, dtype=_I32)
    dum = jnp.full((BNP - BN,), BNP, _I32)
    z480 = jnp.zeros((BNP - BN,), _I32)
    dst_asm = jnp.concatenate([ar, dum, ar, dum])
    srcA_asm = jnp.concatenate([b_x, z480, N_ENT + brel, z480])
    srcB_asm = jnp.concatenate([sx, z480, N_ENT + brel, z480])
    Tc = [jnp.concatenate([pc, rc], axis=0)
          for pc, rc in zip(_chunks(P, 64), _chunks(Rr, 64))]
    xwc = _prop(Tc + Tc, srcA_asm, srcB_asm, dst_asm, BNP, two_src=True)

    p1c = _prop(xwc, src_p, src_p, dst_p, BN)
    inv_deg_b = (1.0 / deg_b)[:, None]
    W2p = jnp.pad(sg1_W2, ((0, 0), (0, 40)))
    hwc = _stage_mce1(p1c, inv_deg_b, W2p)

    g_raw = _prop(list(hwc), src_p, src_p, dst_p, BN)
    g12c = _ew_div(g_raw, inv_deg_b, BN)

    arp = _pad_e(ar, BNP, 0)
    bxp = _pad_e(b_x, BNP, N_ENT)
    o_raw = _prop(g12c, arp, arp, bxp, N_ENT)

    ent_shuf = jnp.take(ent, shuf_idx, axis=0)
    ec, ec_, e12c = _finish(g12c, o_raw, inv_cnt[:, None], S[:, None],
                            leo_W, leo_b, ent, ent_shuf)

    p12_raw = _prop(e12c, gsrc_p, gsrc_p, gdst_p, N_ENT)
    inv_deg_g = (1.0 / deg_g)[:, None]
    W2gp = jnp.pad(sg2_W2, ((0, 0), (0, 40)))
    hbwc = _stage_big(p12_raw, inv_deg_g, sg2_W1, W2gp)
    eg_raw = _prop(hbwc, gsrc_p, gsrc_p, gdst_p, N_ENT)
    eg = _eg_cat(eg_raw[:3], inv_deg_g)
    eg_ = _eg_cat(eg_raw[3:], inv_deg_g)

    def dgi(h1, h2):
        m = _dgi_mean(h1, dgi_W, dgi_b)[0]
        c = jax.nn.sigmoid(m / N_ENT)
        v = dgi_Wd @ c
        sc1 = _dgi_score(h1, dgi_W, dgi_b, v)[:, 0]
        sc2 = _dgi_score(h2, dgi_W, dgi_b, v)[:, 0]
        return jnp.concatenate([sc1, sc2])[None, :]

    local_logits = dgi(ec, ec_)
    global_logits = dgi(eg, eg_)

    h = jnp.take(ec, batch_inputs[:, 0], axis=0)
    r = jnp.take(rel, batch_inputs[:, 1], axis=0)
    t = jnp.take(ec, batch_inputs[:, 2], axis=0)
    out_conv = _convkb(h, r, t, conv_w, conv_b, fc_w.reshape(50, D)) + fc_b
    return (out_conv, local_logits, global_logits)


# speedup vs baseline: 1.0491x; 1.0491x over previous
import functools
import jax
import jax.numpy as jnp
from jax import lax
from jax.experimental import pallas as pl
from jax.experimental.pallas import tpu as pltpu
from jax.experimental.pallas import tpu_sc as plsc

N_ENT = 10000
N_REL = 500
D = 200
BN = 20000
EB = 160000
EG = 160000
B = 4096
NHID = 256
HID = 512

_I32 = jnp.int32
_F32 = jnp.float32


@functools.lru_cache(maxsize=None)
def _make_prop_kernel(nc, C, e_pad, n_dst, two_src):
    EBLK = e_pad // 2048
    n_out = -(-n_dst // 128) * 128
    n_zpad = n_out if n_dst < n_out else n_out + 128
    rz = n_zpad // 16
    ro = n_out // 16
    mesh = plsc.VectorSubcoreMesh(core_axis_name="c", subcore_axis_name="s")
    scratch = [
        pltpu.VMEM((EBLK, 128), _I32),
        pltpu.VMEM((EBLK, 128), _I32),
        pltpu.VMEM((EBLK, 128), _I32),
        pltpu.VMEM((128, C), _F32),
        pltpu.VMEM_SHARED((n_zpad, C), _F32),
        pltpu.SemaphoreType.DMA,
    ]

    @functools.partial(
        pl.kernel, mesh=mesh,
        out_type=jax.ShapeDtypeStruct((nc * n_out, C), _F32),
        compiler_params=pltpu.CompilerParams(use_tc_tiling_on_sc=False),
        scratch_types=scratch)
    def k(*refs):
        tabs = refs[:nc]
        (srcA, srcB, dst2d, zrows, out,
         srcA_v, srcB_v, dst_v, buf, acc, sem) = refs[nc:]
        tid = lax.axis_index("s")
        sc = lax.axis_index("c")
        pltpu.sync_copy(dst2d.at[tid], dst_v)
        pltpu.sync_copy(srcA.at[tid], srcA_v)
        if two_src:
            pltpu.sync_copy(srcB.at[tid], srcB_v)
        for ci in range(nc):
            @pl.when(sc == ci % 2)
            def _(ci=ci):
                pltpu.sync_copy(zrows.at[pl.ds(tid * rz, rz)],
                                acc.at[pl.ds(tid * rz, rz)])
                plsc.subcore_barrier()
                tab = tabs[ci]
                s_v = srcB_v if (two_src and ci >= nc // 2) else srcA_v

                def step(j, carry):
                    pltpu.async_copy(tab.at[s_v.at[j]], buf, sem).wait()
                    pltpu.sync_copy(buf, acc.at[dst_v.at[j]], add=True)
                    return carry

                lax.fori_loop(0, EBLK, step, 0)
                plsc.subcore_barrier()
                pltpu.sync_copy(acc.at[pl.ds(tid * ro, ro)],
                                out.at[pl.ds(ci * n_out + tid * ro, ro)])
                plsc.subcore_barrier()

    return k


def _prop(tabs, srcA, srcB, dst, n_dst, two_src=False):
    nc = len(tabs)
    C = tabs[0].shape[1]
    e_pad = dst.shape[0]
    n_out = -(-n_dst // 128) * 128
    k = _make_prop_kernel(nc, C, e_pad, n_dst, two_src)
    eblk = e_pad // 2048
    srcA2 = srcA.reshape(16, eblk, 128)
    srcB2 = srcB.reshape(16, eblk, 128)
    dst2 = dst.reshape(16, eblk, 128)
    zrows = jnp.zeros((n_out if n_dst < n_out else n_out + 128, C), _F32)
    raw = k(*tabs, srcA2, srcB2, dst2, zrows)
    return [raw[i * n_out:i * n_out + n_dst] for i in range(nc)]


@functools.lru_cache(maxsize=None)
def _make_count_kernel(e_pad, n_dst):
    EBLK = e_pad // 4096
    n_out = -(-n_dst // 128) * 128
    n_zpad = n_out if n_dst < n_out else n_out + 128
    rz = n_zpad // 16
    ro = n_out // 16
    mesh = plsc.VectorSubcoreMesh(core_axis_name="c", subcore_axis_name="s")
    scratch = [
        pltpu.VMEM((EBLK, 128), _I32),
        pltpu.VMEM((128, 16), _F32),
        pltpu.VMEM_SHARED((n_zpad, 16), _F32),
    ]

    @functools.partial(
        pl.kernel, mesh=mesh,
        out_type=jax.ShapeDtypeStruct((2 * n_out, 16), _F32),
        compiler_params=pltpu.CompilerParams(use_tc_tiling_on_sc=False),
        scratch_types=scratch)
    def k(dst2d, zrows, ones_hbm, out, dst_v, buf, acc):
        tid = lax.axis_index("s")
        sc = lax.axis_index("c")
        g = sc * 16 + tid
        pltpu.sync_copy(dst2d.at[g], dst_v)
        pltpu.sync_copy(ones_hbm, buf)
        pltpu.sync_copy(zrows.at[pl.ds(tid * rz, rz)],
                        acc.at[pl.ds(tid * rz, rz)])
        plsc.subcore_barrier()

        def step(j, carry):
            pltpu.sync_copy(buf, acc.at[dst_v.at[j]], add=True)
            return carry

        lax.fori_loop(0, EBLK, step, 0)
        plsc.subcore_barrier()
        pltpu.sync_copy(acc.at[pl.ds(tid * ro, ro)],
                        out.at[pl.ds(sc * n_out + tid * ro, ro)])

    return k


def _seg_count(dst, n_dst):
    E = dst.shape[0]
    e_pad = -(-E // 4096) * 4096
    dstp = jnp.concatenate([dst, jnp.full((e_pad - E,), n_dst, _I32)])
    n_out = -(-n_dst // 128) * 128
    k = _make_count_kernel(e_pad, n_dst)
    ones = jnp.ones((128, 16), _F32)
    zrows = jnp.zeros((n_out if n_dst < n_out else n_out + 128, 16), _F32)
    po = k(dstp.reshape(32, e_pad // 4096, 128), zrows, ones)
    return po[:n_dst, 0] + po[n_out:n_out + n_dst, 0]


def _pad_e(x, e_pad, fill):
    return jnp.concatenate([x, jnp.full((e_pad - x.shape[0],), fill, _I32)])


def _chunks(x, C):
    n, d = x.shape
    return [x[:, i * C:(i + 1) * C] for i in range(d // C)]


def _row_spec(blk, d):
    return pl.BlockSpec((blk, d), lambda i: (i, 0))


def _full_spec(shape):
    return pl.BlockSpec(shape, lambda i: tuple(0 for _ in shape))


def _norm_body(x_ref, o_ref):
    x = x_ref[...]
    n = jnp.sqrt(jnp.sum(x * x, axis=1, keepdims=True))
    o_ref[...] = x / (n + 1e-12)


def _norm(x):
    n, d = x.shape
    blk = min(n, 1000)
    return pl.pallas_call(
        _norm_body, grid=(n // blk,),
        in_specs=[_row_spec(blk, d)],
        out_specs=_row_spec(blk, d),
        out_shape=jax.ShapeDtypeStruct((n, d), _F32),
    )(x)


def _mm_body(x_ref, w_ref, o_ref):
    o_ref[...] = jax.lax.dot(x_ref[...], w_ref[...],
                             precision=jax.lax.Precision.HIGHEST)


def _mm(x, w):
    n, kdim = x.shape
    m = w.shape[1]
    blk = 1000 if n % 1000 == 0 else n
    return pl.pallas_call(
        _mm_body, grid=(n // blk,),
        in_specs=[_row_spec(blk, kdim), _full_spec((kdim, m))],
        out_specs=_row_spec(blk, m),
        out_shape=jax.ShapeDtypeStruct((n, m), _F32),
    )(x, w)


def _stage_mce1_body(c0, c1, c2, c3, c4, c5, c6, c7, inv, w, *outs):
    h1 = jax.nn.relu(jnp.concatenate(
        [c0[...], c1[...], c2[...], c3[...]], axis=1) * inv[...])
    h2 = jax.nn.relu(jnp.concatenate(
        [c4[...], c5[...], c6[...], c7[...]], axis=1) * inv[...])
    o1 = jax.lax.dot(h1, w[...], precision=jax.lax.Precision.HIGHEST)
    o2 = jax.lax.dot(h2, w[...], precision=jax.lax.Precision.HIGHEST)
    for j in range(3):
        outs[j][...] = o1[:, j * 80:(j + 1) * 80]
        outs[3 + j][...] = o2[:, j * 80:(j + 1) * 80]


def _stage_mce1(p1c, inv_deg, W2p):
    blk = 1000
    return pl.pallas_call(
        _stage_mce1_body, grid=(BN // blk,),
        in_specs=[_row_spec(blk, 64)] * 8 + [_row_spec(blk, 1),
                                             _full_spec((NHID, 240))],
        out_specs=[_row_spec(blk, 80)] * 6,
        out_shape=[jax.ShapeDtypeStruct((BN, 80), _F32)] * 6,
    )(*p1c, inv_deg, W2p)


def _ew_div_body(c0, c1, c2, c3, c4, c5, inv, *outs):
    for j, c in enumerate((c0, c1, c2, c3, c4, c5)):
        outs[j][...] = c[...] * inv[...]


def _ew_div(chunks, inv, n):
    blk = 1000
    return list(pl.pallas_call(
        _ew_div_body, grid=(n // blk,),
        in_specs=[_row_spec(blk, 80)] * 6 + [_row_spec(blk, 1)],
        out_specs=[_row_spec(blk, 80)] * 6,
        out_shape=[jax.ShapeDtypeStruct((n, 80), _F32)] * 6,
    )(*chunks, inv))


def _finish_body(g0, g1, g2, g3, g4, g5, o0, o1, o2, o3, o4, o5,
                 invc, s, wa, wb, bvec, base1, base2,
                 ec, ec_, *e12c):
    gg1 = jnp.concatenate([g0[...], g1[...], g2[...]], axis=1)[:, :D]
    gg2 = jnp.concatenate([g3[...], g4[...], g5[...]], axis=1)[:, :D]
    oo1 = jnp.concatenate([o0[...], o1[...], o2[...]], axis=1)[:, :D] * invc[...]
    oo2 = jnp.concatenate([o3[...], o4[...], o5[...]], axis=1)[:, :D] * invc[...]
    hp = jax.lax.Precision.HIGHEST
    e1 = (s[...] * (jax.lax.dot(gg1, wa[...], precision=hp)
                    + jax.lax.dot(oo1, wb[...], precision=hp))
          + bvec[...] + base1[...])
    e2 = (s[...] * (jax.lax.dot(gg2, wa[...], precision=hp)
                    + jax.lax.dot(oo2, wb[...], precision=hp))
          + bvec[...] + base2[...])
    ec[...] = e1
    ec_[...] = e2
    zpad = jnp.zeros((e1.shape[0], 40), _F32)
    e1p = jnp.concatenate([e1, zpad], axis=1)
    e2p = jnp.concatenate([e2, zpad], axis=1)
    for j in range(2):
        e12c[j][...] = e1p[:, j * 120:(j + 1) * 120]
        e12c[2 + j][...] = e2p[:, j * 120:(j + 1) * 120]


def _finish(g12c, o_raw, inv_cnt, S, leo_W, leo_b, ent, ent_shuf):
    blk = 1000
    outs = pl.pallas_call(
        _finish_body, grid=(N_ENT // blk,),
        in_specs=[_row_spec(blk, 80)] * 12
        + [_row_spec(blk, 1), _row_spec(blk, 1),
           _full_spec((D, D)), _full_spec((D, D)),
           pl.BlockSpec((1, D), lambda i: (0, 0)),
           _row_spec(blk, D), _row_spec(blk, D)],
        out_specs=[_row_spec(blk, D)] * 2 + [_row_spec(blk, 120)] * 4,
        out_shape=[jax.ShapeDtypeStruct((N_ENT, D), _F32)] * 2
        + [jax.ShapeDtypeStruct((N_ENT, 120), _F32)] * 4,
    )(*g12c, *o_raw, inv_cnt, S, leo_W[:D], leo_W[D:], leo_b[None, :],
      ent, ent_shuf)
    return outs[0], outs[1], list(outs[2:])


def _stage_big_body(c0, c1, c2, c3, inv, w1, w2, *outs):
    pb1 = (jnp.concatenate([c0[...], c1[...]], axis=1) * inv[...])[:, :D]
    pb2 = (jnp.concatenate([c2[...], c3[...]], axis=1) * inv[...])[:, :D]
    hp = jax.lax.Precision.HIGHEST
    o1 = jax.lax.dot(jax.nn.relu(jax.lax.dot(pb1, w1[...], precision=hp)),
                     w2[...], precision=hp)
    o2 = jax.lax.dot(jax.nn.relu(jax.lax.dot(pb2, w1[...], precision=hp)),
                     w2[...], precision=hp)
    for j in range(2):
        outs[j][...] = o1[:, j * 120:(j + 1) * 120]
        outs[2 + j][...] = o2[:, j * 120:(j + 1) * 120]


def _stage_big(p12_raw, inv_deg_g, W1, W2p):
    blk = 1000
    return list(pl.pallas_call(
        _stage_big_body, grid=(N_ENT // blk,),
        in_specs=[_row_spec(blk, 120)] * 4
        + [_row_spec(blk, 1), _full_spec((D, NHID)), _full_spec((NHID, 240))],
        out_specs=[_row_spec(blk, 120)] * 4,
        out_shape=[jax.ShapeDtypeStruct((N_ENT, 120), _F32)] * 4,
    )(*p12_raw, inv_deg_g, W1, W2p))


def _eg_body(c0, c1, inv, o_ref):
    o_ref[...] = (jnp.concatenate([c0[...], c1[...]], axis=1)
                  * inv[...])[:, :D]


def _eg_cat(chunks, inv_deg_g):
    blk = 1000
    return pl.pallas_call(
        _eg_body, grid=(N_ENT // blk,),
        in_specs=[_row_spec(blk, 120)] * 2 + [_row_spec(blk, 1)],
        out_specs=_row_spec(blk, D),
        out_shape=jax.ShapeDtypeStruct((N_ENT, D), _F32),
    )(*chunks, inv_deg_g)


def _dgi_mean_body(h_ref, w_ref, b_ref, o_ref):
    i = pl.program_id(0)
    hp = jax.lax.Precision.HIGHEST
    e = jax.nn.relu(jax.lax.dot(h_ref[...], w_ref[...], precision=hp)
                    + b_ref[...])
    part = jnp.sum(e, axis=0, keepdims=True)

    @pl.when(i == 0)
    def _():
        o_ref[...] = jnp.zeros_like(o_ref)

    o_ref[...] += part


def _dgi_mean(h, W, b):
    blk = 1000
    return pl.pallas_call(
        _dgi_mean_body, grid=(N_ENT // blk,),
        in_specs=[_row_spec(blk, D), _full_spec((D, HID)),
                  pl.BlockSpec((1, HID), lambda i: (0, 0))],
        out_specs=pl.BlockSpec((1, HID), lambda i: (0, 0)),
        out_shape=jax.ShapeDtypeStruct((1, HID), _F32),
    )(h, W, b[None, :])


def _dgi_score_body(h_ref, w_ref, b_ref, v_ref, o_ref):
    hp = jax.lax.Precision.HIGHEST
    e = jax.nn.relu(jax.lax.dot(h_ref[...], w_ref[...], precision=hp)
                    + b_ref[...])
    o_ref[...] = jax.lax.dot(e, v_ref[...], precision=hp)


def _dgi_score(h, W, b, v):
    blk = 1000
    return pl.pallas_call(
        _dgi_score_body, grid=(N_ENT // blk,),
        in_specs=[_row_spec(blk, D), _full_spec((D, HID)),
                  pl.BlockSpec((1, HID), lambda i: (0, 0)),
                  _full_spec((HID, 1))],
        out_specs=_row_spec(blk, 1),
        out_shape=jax.ShapeDtypeStruct((N_ENT, 1), _F32),
    )(h, W, b[None, :], v[:, None])


def _convkb_body(h_ref, r_ref, t_ref, cw_ref, cb_ref, fc2_ref, o_ref):
    h = h_ref[...]
    r = r_ref[...]
    t = t_ref[...]
    acc = jnp.zeros_like(h)
    for o in range(50):
        co = jax.nn.relu(cw_ref[o, 0] * h + cw_ref[o, 1] * r + cw_ref[o, 2] * t
                         + cb_ref[o])
        acc = acc + co * fc2_ref[o, :][None, :]
    o_ref[...] = jnp.sum(acc, axis=1, keepdims=True)


def _convkb(h, r, t, conv_w, conv_b, fc2):
    blk = 1024
    return pl.pallas_call(
        _convkb_body,
        grid=(B // blk,),
        in_specs=[
            pl.BlockSpec((blk, D), lambda i: (i, 0)),
            pl.BlockSpec((blk, D), lambda i: (i, 0)),
            pl.BlockSpec((blk, D), lambda i: (i, 0)),
            pl.BlockSpec((50, 3), lambda i: (0, 0), memory_space=pltpu.SMEM),
            pl.BlockSpec((50,), lambda i: (0,), memory_space=pltpu.SMEM),
            pl.BlockSpec((50, D), lambda i: (0, 0)),
        ],
        out_specs=pl.BlockSpec((blk, 1), lambda i: (i, 0)),
        out_shape=jax.ShapeDtypeStruct((B, 1), jnp.float32),
    )(h, r, t, conv_w, conv_b, fc2)


def kernel(*args):
    with jax.default_matmul_precision("float32"):
        return _kernel_impl(*args)


def _kernel_impl(entity_embeddings, relation_embeddings, sg1_W1, sg1_W2, sg2_W1,
                 sg2_W2, le_W, le_b, leo_W, leo_b, dgi_W, dgi_b, dgi_Wd,
                 conv_w, conv_b, fc_w, fc_b,
                 b_x, b_node_graph_index, b_edge_index, big_edge_index,
                 batch_inputs, shuf_idx):
    ent = _norm(entity_embeddings)
    rel = _norm(relation_embeddings)

    b_x = b_x.astype(_I32)
    brel = b_node_graph_index.astype(_I32)
    src = b_edge_index[0].astype(_I32)
    dst = b_edge_index[1].astype(_I32)
    gsrc = big_edge_index[0].astype(_I32)
    gdst = big_edge_index[1].astype(_I32)
    shuf_idx = shuf_idx.astype(_I32)

    deg_b = jnp.maximum(_seg_count(dst, BN), 1.0)
    deg_g = jnp.maximum(_seg_count(gdst, N_ENT), 1.0)
    cnt = _seg_count(b_x, N_ENT)
    S = cnt / (cnt + 1e-16)
    inv_cnt = 1.0 / jnp.maximum(cnt, 1.0)

    eb_pad = -(-EB // 2048) * 2048
    src_p = _pad_e(src, eb_pad, 0)
    dst_p = _pad_e(dst, eb_pad, BN)
    gsrc_p = _pad_e(gsrc, eb_pad, 0)
    gdst_p = _pad_e(gdst, eb_pad, N_ENT)

    P = _mm(ent, sg1_W1[:D])
    Rr = _mm(rel, sg1_W1[D:])
    sx = jnp.take(shuf_idx, b_x)
    BNP = BN + 480
    ar = jnp.arange(BN, dtype=_I32)
    dum = jnp.full((BNP - BN,), BNP, _I32)
    z480 = jnp.zeros((BNP - BN,), _I32)
    dst_asm = jnp.concatenate([ar, dum, ar, dum])
    srcA_asm = jnp.concatenate([b_x, z480, N_ENT + brel, z480])
    srcB_asm = jnp.concatenate([sx, z480, N_ENT + brel, z480])
    Tc = [jnp.concatenate([pc, rc], axis=0)
          for pc, rc in zip(_chunks(P, 64), _chunks(Rr, 64))]
    xwc = _prop(Tc + Tc, srcA_asm, srcB_asm, dst_asm, BNP, two_src=True)

    p1c = _prop(xwc, src_p, src_p, dst_p, BN)
    inv_deg_b = (1.0 / deg_b)[:, None]
    W2p = jnp.pad(sg1_W2, ((0, 0), (0, 40)))
    hwc = _stage_mce1(p1c, inv_deg_b, W2p)

    g_raw = _prop(list(hwc), src_p, src_p, dst_p, BN)
    g12c = _ew_div(g_raw, inv_deg_b, BN)

    arp = _pad_e(ar, BNP, 0)
    bxp = _pad_e(b_x, BNP, N_ENT)
    o_raw = _prop(g12c, arp, arp, bxp, N_ENT)

    ent_shuf = jnp.take(ent, shuf_idx, axis=0)
    ec, ec_, e12c = _finish(g12c, o_raw, inv_cnt[:, None], S[:, None],
                            leo_W, leo_b, ent, ent_shuf)

    p12_raw = _prop(e12c, gsrc_p, gsrc_p, gdst_p, N_ENT)
    inv_deg_g = (1.0 / deg_g)[:, None]
    W2gp = jnp.pad(sg2_W2, ((0, 0), (0, 40)))
    hbwc = _stage_big(p12_raw, inv_deg_g, sg2_W1, W2gp)
    eg_raw = _prop(hbwc, gsrc_p, gsrc_p, gdst_p, N_ENT)
    eg = _eg_cat(eg_raw[:2], inv_deg_g)
    eg_ = _eg_cat(eg_raw[2:], inv_deg_g)

    def dgi(h1, h2):
        m = _dgi_mean(h1, dgi_W, dgi_b)[0]
        c = jax.nn.sigmoid(m / N_ENT)
        v = dgi_Wd @ c
        sc1 = _dgi_score(h1, dgi_W, dgi_b, v)[:, 0]
        sc2 = _dgi_score(h2, dgi_W, dgi_b, v)[:, 0]
        return jnp.concatenate([sc1, sc2])[None, :]

    local_logits = dgi(ec, ec_)
    global_logits = dgi(eg, eg_)

    h = jnp.take(ec, batch_inputs[:, 0], axis=0)
    r = jnp.take(rel, batch_inputs[:, 1], axis=0)
    t = jnp.take(ec, batch_inputs[:, 2], axis=0)
    out_conv = _convkb(h, r, t, conv_w, conv_b, fc_w.reshape(50, D)) + fc_b
    return (out_conv, local_logits, global_logits)
